# baseline (speedup 1.0000x reference)
import jax
import jax.numpy as jnp
from jax.experimental import pallas as pl
from jax.experimental.pallas import tpu as pltpu

HP = jax.lax.Precision.HIGHEST
NB = 25
K = 256


def _conv(x, w, b, stride, pad):
    y = jax.lax.conv_general_dilated(
        x, w, (stride, stride), [(pad, pad), (pad, pad)],
        dimension_numbers=('NCHW', 'OIHW', 'NCHW'))
    return y + b[None, :, None, None]


def _heat_body(xs_ref, img_ref, out_ref):
    xs = xs_ref[0]
    rmin = jnp.min(xs, axis=-1, keepdims=True)
    rmax = jnp.max(xs, axis=-1, keepdims=True)
    xn = (xs - rmin) / (rmax - rmin)

    a = img_ref[0]
    wl = jnp.maximum(jnp.maximum(a, jnp.roll(a, -1, axis=1)),
                     jnp.maximum(jnp.roll(a, -2, axis=1),
                                 jnp.roll(a, -3, axis=1)))
    selL = jnp.where(jax.lax.broadcasted_iota(jnp.int32, (224, 56), 0)
                     == 4 * jax.lax.broadcasted_iota(jnp.int32, (224, 56), 1),
                     1.0, 0.0)
    p1 = jax.lax.dot(wl, selL, precision=HP)
    ws = jnp.maximum(jnp.maximum(p1, jnp.roll(p1, -1, axis=0)),
                     jnp.maximum(jnp.roll(p1, -2, axis=0),
                                 jnp.roll(p1, -3, axis=0)))
    selS = jnp.where(jax.lax.broadcasted_iota(jnp.int32, (56, 224), 1)
                     == 4 * jax.lax.broadcasted_iota(jnp.int32, (56, 224), 0),
                     1.0, 0.0)
    imgp = jax.lax.dot(selS, ws, precision=HP)
    imin = jnp.min(imgp, axis=-1, keepdims=True)
    out_ref[0] = xn * (imgp - imin)


def _topk_body(h_ref, out_ref):
    h = h_ref[0]
    bits = jax.lax.bitcast_convert_type(h, jnp.int32)
    t = jnp.int32(0)
    for bit in range(30, -1, -1):
        tc = t + (1 << bit)
        ge = jnp.sum(jnp.where(bits >= tc, 1, 0))
        t = jnp.where(ge >= K, tc, t)
    m = jnp.sum(jnp.where(bits > t, 1, 0))
    r = K - m
    eqf = jnp.where(bits == t, 1.0, 0.0)
    gtf = jnp.where(bits > t, 1.0, 0.0)
    U = jnp.where(jax.lax.broadcasted_iota(jnp.int32, (128, 128), 0)
                  <= jax.lax.broadcasted_iota(jnp.int32, (128, 128), 1),
                  1.0, 0.0)
    Ls = jnp.where(jax.lax.broadcasted_iota(jnp.int32, (NB, NB), 0)
                   > jax.lax.broadcasted_iota(jnp.int32, (NB, NB), 1),
                   1.0, 0.0)

    def excl_cumsum(x):
        within = jax.lax.dot(x, U, precision=HP)
        carry = jax.lax.dot(Ls, within[:, 127:128], precision=HP)
        return within + carry - x

    eq_excl = excl_cumsum(eqf)
    eq_sel = eqf * jnp.where(eq_excl < r.astype(jnp.float32), 1.0, 0.0)
    sel = gtf + eq_sel
    pos = excl_cumsum(sel)
    iota_s = jax.lax.broadcasted_iota(jnp.int32, (K, 1), 0).astype(jnp.float32)
    idxf = (jax.lax.broadcasted_iota(jnp.int32, (NB, 128), 0) * 128
            + jax.lax.broadcasted_iota(jnp.int32, (NB, 128), 1)
            ).astype(jnp.float32)
    acc = jnp.zeros((K, 2), jnp.float32)
    for j in range(NB):
        oh = jnp.where((iota_s == pos[j:j + 1, :]) & (sel[j:j + 1, :] > 0.5),
                       1.0, 0.0)
        payload = jnp.concatenate([h[j:j + 1, :], idxf[j:j + 1, :]], axis=0)
        acc = acc + jax.lax.dot_general(oh, payload,
                                        (((1,), (1,)), ((), ())),
                                        precision=HP)
    val = acc[:, 0:1]
    cidx = acc[:, 1:2]
    valT = jnp.transpose(val)
    gt = jnp.where(valT > val, 1.0, 0.0)
    eq2 = jnp.where((valT == val) & (jnp.transpose(cidx) < cidx), 1.0, 0.0)
    rank = jax.lax.dot_general(gt + eq2, jnp.ones((K, 1), jnp.float32),
                               (((1,), (0,)), ((), ())), precision=HP)
    idxi = cidx.astype(jnp.int32)
    payload2 = jnp.concatenate(
        [(idxi % 56).astype(jnp.float32), (idxi // 56).astype(jnp.float32),
         jnp.zeros((K, 1), jnp.float32), val], axis=1)
    perm = jnp.where(iota_s == jnp.transpose(rank), 1.0, 0.0)
    out_ref[0] = jax.lax.dot(perm, payload2, precision=HP)


def kernel(img, W1, b1, W2, b2, Wc, bc):
    x = jax.nn.relu(_conv(img, W1, b1, 2, 3))
    x = jax.nn.relu(_conv(x, W2, b2, 2, 1))
    x = _conv(x, Wc, bc, 1, 0)
    x = jax.nn.sigmoid(x)
    heat = pl.pallas_call(
        _heat_body,
        grid=(8,),
        in_specs=[pl.BlockSpec((1, 56, 56), lambda b: (b, 0, 0)),
                  pl.BlockSpec((1, 224, 224), lambda b: (b, 0, 0))],
        out_specs=pl.BlockSpec((1, 56, 56), lambda b: (b, 0, 0)),
        out_shape=jax.ShapeDtypeStruct((8, 56, 56), jnp.float32),
    )(x.reshape(8, 56, 56), img[:, 0])
    hpad = jnp.pad(heat.reshape(8, 3136), ((0, 0), (0, NB * 128 - 3136)))
    return pl.pallas_call(
        _topk_body,
        grid=(8,),
        in_specs=[pl.BlockSpec((1, NB, 128), lambda b: (b, 0, 0))],
        out_specs=pl.BlockSpec((1, K, 4), lambda b: (b, 0, 0)),
        out_shape=jax.ShapeDtypeStruct((8, K, 4), jnp.float32),
    )(hpad.reshape(8, NB, 128))


# speedup vs baseline: 1.0368x; 1.0368x over previous
import jax
import jax.numpy as jnp
from jax.experimental import pallas as pl
from jax.experimental.pallas import tpu as pltpu

HP = jax.lax.Precision.HIGHEST
K = 256
R = 56
C = 56


def _conv(x, w, b, stride, pad):
    y = jax.lax.conv_general_dilated(
        x, w, (stride, stride), [(pad, pad), (pad, pad)],
        dimension_numbers=('NCHW', 'OIHW', 'NCHW'))
    return y + b[None, :, None, None]


def _iota2(shape, dim):
    return jax.lax.broadcasted_iota(jnp.int32, shape, dim)


def _tr(v, n):
    eye = jnp.where(_iota2((n, n), 0) == _iota2((n, n), 1), 1.0, 0.0)
    return jax.lax.dot_general(v, eye, (((0,), (0,)), ((), ())), precision=HP)


def _body(xs_ref, img_ref, out_ref):
    xs = xs_ref[0]
    rmin = jnp.min(xs, axis=-1, keepdims=True)
    rmax = jnp.max(xs, axis=-1, keepdims=True)
    xn = (xs - rmin) / (rmax - rmin)

    a = img_ref[0]
    wl = jnp.maximum(jnp.maximum(a, jnp.roll(a, -1, axis=1)),
                     jnp.maximum(jnp.roll(a, -2, axis=1),
                                 jnp.roll(a, -3, axis=1)))
    selL = jnp.where(_iota2((224, C), 0) == 4 * _iota2((224, C), 1), 1.0, 0.0)
    p1 = jax.lax.dot(wl, selL, precision=HP)
    ws = jnp.maximum(jnp.maximum(p1, jnp.roll(p1, -1, axis=0)),
                     jnp.maximum(jnp.roll(p1, -2, axis=0),
                                 jnp.roll(p1, -3, axis=0)))
    selS = jnp.where(_iota2((R, 224), 1) == 4 * _iota2((R, 224), 0), 1.0, 0.0)
    imgp = jax.lax.dot(selS, ws, precision=HP)
    imin = jnp.min(imgp, axis=-1, keepdims=True)
    h = xn * (imgp - imin)

    bits = jax.lax.bitcast_convert_type(h, jnp.int32)
    t = jnp.int32(0)
    for bit in range(29, -1, -1):
        tc = t + (1 << bit)
        ge = jnp.sum(jnp.where(bits >= tc, 1, 0))
        t = jnp.where(ge >= K, tc, t)
    m = jnp.sum(jnp.where(bits > t, 1, 0))
    r_need = (K - m).astype(jnp.float32)
    eqf = jnp.where(bits == t, 1.0, 0.0)
    gtf = jnp.where(bits > t, 1.0, 0.0)

    U = jnp.where(_iota2((R, C), 0) <= _iota2((R, C), 1), 1.0, 0.0)
    Lst = jnp.where(_iota2((R, R), 0) > _iota2((R, R), 1), 1.0, 0.0)
    onesC = jnp.ones((C, 1), jnp.float32)

    def excl_cumsum(x):
        within = jax.lax.dot(x, U, precision=HP)
        carry = jax.lax.dot(Lst, within[:, C - 1:C], precision=HP)
        return within + carry - x

    eq_excl = excl_cumsum(eqf)
    sel = gtf + eqf * jnp.where(eq_excl < r_need, 1.0, 0.0)
    rowpos = jax.lax.dot(sel, U, precision=HP) - sel
    cnt = jax.lax.dot(sel, onesC, precision=HP)
    start = jax.lax.dot(Lst, cnt, precision=HP)

    kio = _iota2((K, 1), 0).astype(jnp.float32)
    startT = _tr(start, R)
    cntT = _tr(cnt, R)
    row1h = jnp.where((kio >= startT) & (kio < startT + cntT), 1.0, 0.0)
    big = jax.lax.dot(
        row1h, jnp.concatenate([sel * rowpos, sel * h, sel], axis=1),
        precision=HP)
    mpos, mval, msel = big[:, :C], big[:, C:2 * C], big[:, 2 * C:]
    startk = jax.lax.dot(row1h, start, precision=HP)
    riota = _iota2((R, 1), 0).astype(jnp.float32)
    rowidx = jax.lax.dot(row1h, riota, precision=HP)
    col1h = msel * jnp.where(mpos == kio - startk, 1.0, 0.0)
    ciota = _iota2((C, 1), 0).astype(jnp.float32)
    val = jax.lax.dot(col1h * mval, onesC, precision=HP)
    colidx = jax.lax.dot(col1h, ciota, precision=HP)

    flat = rowidx * float(C) + colidx
    valT = _tr(val, K)
    gt = jnp.where(valT > val, 1.0, 0.0)
    eq2 = jnp.where((valT == val) & (_tr(flat, K) < flat), 1.0, 0.0)
    rank = jax.lax.dot(gt + eq2, jnp.ones((K, 1), jnp.float32), precision=HP)
    perm = jnp.where(kio == _tr(rank, K), 1.0, 0.0)
    payload = jnp.concatenate(
        [colidx, rowidx, jnp.zeros((K, 1), jnp.float32), val], axis=1)
    out_ref[0] = jax.lax.dot(perm, payload, precision=HP)


def kernel(img, W1, b1, W2, b2, Wc, bc):
    x = jax.nn.relu(_conv(img, W1, b1, 2, 3))
    x = jax.nn.relu(_conv(x, W2, b2, 2, 1))
    x = _conv(x, Wc, bc, 1, 0)
    x = jax.nn.sigmoid(x)
    return pl.pallas_call(
        _body,
        grid=(8,),
        in_specs=[pl.BlockSpec((1, R, C), lambda b: (b, 0, 0)),
                  pl.BlockSpec((1, 224, 224), lambda b: (b, 0, 0))],
        out_specs=pl.BlockSpec((1, K, 4), lambda b: (b, 0, 0)),
        out_shape=jax.ShapeDtypeStruct((8, K, 4), jnp.float32),
    )(x.reshape(8, R, C), img[:, 0])


# speedup vs baseline: 1.1374x; 1.0971x over previous
import jax
import jax.numpy as jnp
from jax.experimental import pallas as pl
from jax.experimental.pallas import tpu as pltpu

HP = jax.lax.Precision.HIGHEST
B = 8
R = 56
C = 56
G = B * R
K = 256
KK = B * K
IH = 224
GI = B * IH


def _conv(x, w, b, stride, pad):
    y = jax.lax.conv_general_dilated(
        x, w, (stride, stride), [(pad, pad), (pad, pad)],
        dimension_numbers=('NCHW', 'OIHW', 'NCHW'))
    return y + b[None, :, None, None]


def _iota(shape, dim):
    return jax.lax.broadcasted_iota(jnp.int32, shape, dim)


def _body(xs_ref, img_ref, out_ref):
    f32 = jnp.float32
    xs = xs_ref[...]
    rmin = jnp.min(xs, axis=-1, keepdims=True)
    rmax = jnp.max(xs, axis=-1, keepdims=True)
    xn = (xs - rmin) / (rmax - rmin)

    a = img_ref[...]
    wl = jnp.maximum(jnp.maximum(a, jnp.roll(a, -1, axis=1)),
                     jnp.maximum(jnp.roll(a, -2, axis=1),
                                 jnp.roll(a, -3, axis=1)))
    selL = jnp.where(_iota((IH, C), 0) == 4 * _iota((IH, C), 1), 1.0, 0.0)
    p1 = jax.lax.dot(wl, selL, precision=HP)
    ws = jnp.maximum(jnp.maximum(p1, jnp.roll(p1, -1, axis=0)),
                     jnp.maximum(jnp.roll(p1, -2, axis=0),
                                 jnp.roll(p1, -3, axis=0)))
    rowsel = jnp.where(_iota((G, GI), 1) == 4 * _iota((G, GI), 0), 1.0, 0.0)
    imgp = jax.lax.dot(rowsel, ws, precision=HP)
    imin = jnp.min(imgp, axis=-1, keepdims=True)
    h = xn * (imgp - imin)

    bits3 = jax.lax.bitcast_convert_type(h, jnp.int32).reshape(B, R, C)
    t = jnp.zeros((B, 1, 1), jnp.int32)
    for bit in range(29, -1, -1):
        tc = t + (1 << bit)
        ge = jnp.sum(jnp.where(bits3 >= tc, 1, 0),
                     axis=2, keepdims=True).sum(axis=1, keepdims=True)
        t = jnp.where(ge >= K, tc, t)
    m = jnp.sum(jnp.where(bits3 > t, 1, 0),
                axis=2, keepdims=True).sum(axis=1, keepdims=True)
    r_need = (K - m).astype(f32)
    eqf = jnp.where(bits3 == t, 1.0, 0.0).reshape(G, C)
    gtf = jnp.where(bits3 > t, 1.0, 0.0).reshape(G, C)
    rneedR = jnp.broadcast_to(r_need, (B, R, 1)).reshape(G, 1)

    U = jnp.where(_iota((C, C), 0) <= _iota((C, C), 1), 1.0, 0.0)
    samerow = _iota((G, G), 0) // R == _iota((G, G), 1) // R
    Lbd = jnp.where(samerow & (_iota((G, G), 0) > _iota((G, G), 1)), 1.0, 0.0)
    onesC = jnp.ones((C, 1), f32)

    within_eq = jax.lax.dot(eqf, U, precision=HP)
    carry_eq = jax.lax.dot(Lbd, within_eq[:, C - 1:C], precision=HP)
    eq_excl = within_eq + carry_eq - eqf
    sel = gtf + eqf * jnp.where(eq_excl < rneedR, 1.0, 0.0)
    rowpos = jax.lax.dot(sel, U, precision=HP) - sel
    cnt = jax.lax.dot(sel, onesC, precision=HP)
    start = jax.lax.dot(Lbd, cnt, precision=HP)

    X = jnp.concatenate([sel * rowpos, sel * h, sel], axis=1)
    XT = jnp.transpose(X)
    I448 = jnp.where(_iota((G, G), 0) == _iota((G, G), 1), 1.0, 0.0)
    startT = jax.lax.dot_general(start, I448, (((0,), (0,)), ((), ())),
                                 precision=HP)
    cntT = jax.lax.dot_general(cnt, I448, (((0,), (0,)), ((), ())),
                               precision=HP)
    kmodC = (_iota((KK, 1), 0) % K).astype(f32)
    ksmpC = _iota((KK, 1), 0) // K
    gsmpT = _iota((1, G), 1) // R
    row1h = jnp.where((ksmpC == gsmpT) & (kmodC >= startT)
                      & (kmodC < startT + cntT), 1.0, 0.0)
    kmodT = (_iota((1, KK), 1) % K).astype(f32)
    ksmpT = _iota((1, KK), 1) // K
    gsmpC = _iota((G, 1), 0) // R
    row1hT = jnp.where((gsmpC == ksmpT) & (kmodT >= start)
                       & (kmodT < start + cnt), 1.0, 0.0)

    big = jax.lax.dot(row1h, X, precision=HP)
    bigT = jax.lax.dot(XT, row1hT, precision=HP)
    mpos, mval, msel = big[:, :C], big[:, C:2 * C], big[:, 2 * C:]
    mposT, mvalT, mselT = bigT[:C], bigT[C:2 * C], bigT[2 * C:]
    startk = jax.lax.dot(row1h, start, precision=HP)
    startkT = jax.lax.dot(startT, row1hT, precision=HP)
    rloc = (_iota((G, 1), 0) % R).astype(f32)
    rlocT = (_iota((1, G), 1) % R).astype(f32)
    rowidx = jax.lax.dot(row1h, rloc, precision=HP)
    rowidxT = jax.lax.dot(rlocT, row1hT, precision=HP)

    col1h = msel * jnp.where(mpos == kmodC - startk, 1.0, 0.0)
    col1hT = mselT * jnp.where(mposT == kmodT - startkT, 1.0, 0.0)
    ciota = _iota((C, 1), 0).astype(f32)
    ciotaT = _iota((1, C), 1).astype(f32)
    val = jax.lax.dot(col1h * mval, onesC, precision=HP)
    valT = jax.lax.dot(jnp.ones((1, C), f32), col1hT * mvalT,
                       precision=HP)
    colidx = jax.lax.dot(col1h, ciota, precision=HP)
    colidxT = jax.lax.dot(ciotaT, col1hT, precision=HP)
    flat = rowidx * float(C) + colidx
    flatT = rowidxT * float(C) + colidxT

    val3 = val.reshape(B, K, 1)
    flat3 = flat.reshape(B, K, 1)
    valT3 = valT.reshape(B, 1, K)
    flatT3 = flatT.reshape(B, 1, K)
    colT3 = colidxT.reshape(B, 1, K)
    rowT3 = rowidxT.reshape(B, 1, K)
    beats = jnp.where((val3 > valT3)
                      | ((val3 == valT3) & (flat3 < flatT3)), 1.0, 0.0)
    rankL = jnp.sum(beats, axis=1, keepdims=True)
    kio3 = _iota((1, K, 1), 1).astype(f32)
    perm = jnp.where(kio3 == rankL, 1.0, 0.0)
    outc = jnp.sum(perm * colT3, axis=2, keepdims=True)
    outr = jnp.sum(perm * rowT3, axis=2, keepdims=True)
    outv = jnp.sum(perm * valT.reshape(B, 1, K), axis=2, keepdims=True)
    out_ref[...] = jnp.concatenate(
        [outc, outr, jnp.zeros((B, K, 1), f32), outv], axis=2)


def kernel(img, W1, b1, W2, b2, Wc, bc):
    x = jax.nn.relu(_conv(img, W1, b1, 2, 3))
    x = jax.nn.relu(_conv(x, W2, b2, 2, 1))
    x = _conv(x, Wc, bc, 1, 0)
    x = jax.nn.sigmoid(x)
    return pl.pallas_call(
        _body,
        out_shape=jax.ShapeDtypeStruct((B, K, 4), jnp.float32),
    )(x.reshape(G, C), img[:, 0].reshape(GI, IH))
